# baseline (speedup 1.0000x reference)
import functools

import jax
import jax.numpy as jnp
from jax import lax
from jax.experimental import pallas as pl
from jax.experimental.pallas import tpu as pltpu
from jax.experimental.pallas import tpu_sc as plsc

NC = 2
NS = 16
NW = NC * NS

RING = 4
PREF = 2


def _gather_call(n_samples, seq, dim):
    chunk = seq // 2
    n_chunks_total = n_samples * 2
    assert n_chunks_total % NW == 0
    n_chunks = n_chunks_total // NW
    assert n_chunks % RING == 0 and PREF <= RING

    mesh = plsc.VectorSubcoreMesh(core_axis_name="c", subcore_axis_name="s")

    @functools.partial(
        pl.kernel,
        mesh=mesh,
        out_type=jax.ShapeDtypeStruct((n_samples, seq, dim), jnp.float32),
        scratch_types=[
            pltpu.VMEM((n_chunks, chunk), jnp.int32),
            pltpu.VMEM((RING, chunk, dim), jnp.float32),
            pltpu.SemaphoreType.DMA((RING,)),
            pltpu.SemaphoreType.DMA((RING,)),
        ],
        compiler_params=pltpu.CompilerParams(use_tc_tiling_on_sc=False),
    )
    def k(table_hbm, idx_hbm, out_hbm, idx_v, rows_v, gsem, osem):
        wid = lax.axis_index("s") * NC + lax.axis_index("c")
        row0 = wid * n_chunks
        pltpu.sync_copy(idx_hbm.at[pl.ds(row0, n_chunks)], idx_v)

        def gather(g, b):
            return pltpu.make_async_copy(
                table_hbm.at[idx_v.at[g]], rows_v.at[b], gsem.at[b])

        def write(g, b):
            r = row0 + g
            i = r // 2
            j0 = (r % 2) * chunk
            return pltpu.make_async_copy(
                rows_v.at[b], out_hbm.at[i, pl.ds(j0, chunk)], osem.at[b])

        for g in range(PREF):
            gather(g, g % RING).start()

        def outer(t, _):
            g0 = t * RING
            for j in range(RING):
                g = g0 + j
                bp = (j + PREF) % RING
                gp = g + PREF

                @pl.when(gp < n_chunks)
                def _issue():
                    @pl.when(gp >= RING)
                    def _drain():
                        write(g, bp).wait()
                    gather(gp, bp).start()

                gather(g, j).wait()
                write(g, j).start()
            return 0

        lax.fori_loop(0, n_chunks // RING, outer, 0)
        for j in range(RING):
            write(j, j).wait()

    return k


def kernel(embedding, x):
    dim = embedding.shape[-1]
    n_samples, seq = x.shape
    idx2d = x.astype(jnp.int32).reshape(n_samples * 2, seq // 2)
    return _gather_call(n_samples, seq, dim)(embedding, idx2d)
